# baseline (speedup 1.0000x reference)
import functools

import jax
import jax.numpy as jnp
from jax import lax
from jax.experimental import pallas as pl
from jax.experimental.pallas import tpu as pltpu
from jax.experimental.pallas import tpu_sc as plsc

B = 32
C = 96
S = 9216
K = 1024
LA = 8
SP = S + 128
BB = 2


def _sortable(bits):
    return bits ^ ((bits >> 31) & jnp.int32(0x7FFFFFFF))


def _tc_body(f_ref, la_ref, wa_ref, out_ref, sp_ref):
    keys = []
    for bb in range(BB):
        la = la_ref[bb]
        x_row = jnp.max(la, axis=0, keepdims=True)

        fs = f_ref[bb] * x_row
        fa = jnp.concatenate([fs, x_row], axis=0)
        out_ref[bb] = lax.dot_general(
            wa_ref[...], fa,
            dimension_numbers=(((1,), (0,)), ((), ())),
            preferred_element_type=jnp.float32,
        )

        xc = jnp.where(x_row == 0.0, jnp.float32(0.0), x_row)
        sp_ref[bb, 0, pl.ds(0, S)] = xc[0]
        keys.append(_sortable(
            lax.bitcast_convert_type(xc.reshape(72, 128), jnp.int32)))

    def search(it, carry):
        inc = lax.shift_left(jnp.int32(1), jnp.int32(31) - it)
        new = []
        for bb in range(BB):
            cand = carry[bb] + inc
            cnt = jnp.sum((keys[bb] >= cand).astype(jnp.int32))
            new.append(jnp.where(cnt >= K, cand, carry[bb]))
        return tuple(new)

    tmin = jnp.int32(-2147483648)
    tstars = lax.fori_loop(0, 32, search, (tmin,) * BB)

    lane = lax.broadcasted_iota(jnp.int32, (1, 128), 1)
    for bb in range(BB):
        tstar = tstars[bb]
        n_gt = jnp.sum((keys[bb] > tstar).astype(jnp.int32))
        th_f = lax.bitcast_convert_type(_sortable(tstar), jnp.float32)
        ngt_f = lax.bitcast_convert_type(n_gt, jnp.float32)
        pack = jnp.where(lane < 16, th_f, ngt_f)
        sp_ref[bb, 0, pl.ds(S, 128)] = pack[0]


def _tc_call(f3, la3, wa):
    return pl.pallas_call(
        _tc_body,
        grid=(B // BB,),
        in_specs=[
            pl.BlockSpec((BB, C, S), lambda i: (i, 0, 0)),
            pl.BlockSpec((BB, LA, S), lambda i: (i, 0, 0)),
            pl.BlockSpec((C, C + 1), lambda i: (0, 0)),
        ],
        out_specs=[
            pl.BlockSpec((BB, C, S), lambda i: (i, 0, 0)),
            pl.BlockSpec((BB, 1, SP), lambda i: (i, 0, 0)),
        ],
        out_shape=[
            jax.ShapeDtypeStruct((B, C, S), jnp.float32),
            jax.ShapeDtypeStruct((B, 1, SP), jnp.float32),
        ],
        compiler_params=pltpu.CompilerParams(
            dimension_semantics=("parallel",)),
    )(f3, la3, wa)


@functools.lru_cache(maxsize=1)
def _make_sc_topk():
    mesh = plsc.VectorSubcoreMesh(core_axis_name="c", subcore_axis_name="s")
    n_chunks = S // 16

    @functools.partial(
        pl.kernel,
        mesh=mesh,
        out_type=jax.ShapeDtypeStruct((B, K), jnp.int32),
        scratch_types=[
            pltpu.VMEM((SP,), jnp.float32),
            pltpu.VMEM((K,), jnp.int32),
        ],
        compiler_params=pltpu.CompilerParams(needs_layout_passes=False),
    )
    def topk(sp_hbm, out_hbm, sc_v, idx_v):
        cid = lax.axis_index("c")
        sid = lax.axis_index("s")
        wid = sid * 2 + cid

        pltpu.sync_copy(sp_hbm.at[wid], sc_v)

        thr = sc_v[pl.ds(S, 16)]
        ngt = plsc.bitcast(sc_v[pl.ds(S + 16, 16)], jnp.int32)
        need_eq = jnp.int32(K) - ngt
        lane = lax.iota(jnp.int32, 16)

        def body(v, carry):
            off, eq_seen = carry
            scv = sc_v[pl.ds(v * 16, 16)]
            gt = scv > thr
            eq = scv == thr
            eqc = plsc.cumsum(eq.astype(jnp.int32))
            sel = jnp.logical_and(eq, (eqc + eq_seen) <= need_eq)
            keep = jnp.logical_or(gt, sel)
            pos = off + plsc.cumsum(keep.astype(jnp.int32)) - 1
            idx = lane + v * 16
            plsc.store_scatter(idx_v, [pos], idx, mask=keep)
            off = off + plsc.all_reduce_population_count(keep)
            eq_seen = eq_seen + plsc.all_reduce_population_count(sel)
            return off, eq_seen

        zeros = jnp.zeros((16,), jnp.int32)
        lax.fori_loop(0, n_chunks, body, (zeros, zeros))
        pltpu.sync_copy(idx_v, out_hbm.at[wid])

    return topk


@jax.jit
def kernel(feature, la_outs, W, b):
    f3 = feature.reshape(B, C, S)
    la3 = la_outs.reshape(B, LA, S)
    wa = jnp.concatenate([W, b[:, None]], axis=1)

    out3, sp = _tc_call(f3, la3, wa)
    keep_index = _make_sc_topk()(sp.reshape(B, SP))
    return out3.swapaxes(1, 2), keep_index
